# baseline (speedup 1.0000x reference)
import functools

import jax
import jax.numpy as jnp
from jax.experimental import pallas as pl

B = 4096
HD = 256
ED = 256
NSLOT = 65
NLOOK = 66
MAX_LEN = 25
EPS = 1e-20

BB = 64


def _body(h_ref, E_ref, nent_ref, edist_ref, null_ref, look_ref, et_ref,
          eidx_ref, fin_ref, lam_ref, Wr_ref, Wec_ref, Wlh_ref, Wle_ref,
          bL_ref, Wef_ref, Wei_ref, Wx_ref, Wxn_ref,
          oet_ref, oeidx_ref, oelen_ref, ox_ref, E2_ref, onent_ref,
          od_ref, onc_ref, olook_ref):
    h = h_ref[...]
    E = E_ref[...].reshape(BB, NSLOT, ED)
    e_idx1 = eidx_ref[...] + 1
    em = (et_ref[...] == 1).astype(jnp.float32)
    fin = fin_ref[...].astype(jnp.float32)
    lam = lam_ref[0, 0]

    iota65 = jax.lax.broadcasted_iota(jnp.int32, (BB, NSLOT), 1)
    onehot = (iota65 == e_idx1).astype(jnp.float32)

    curr = jnp.sum(E * onehot[:, :, None], axis=1)

    prec = jax.lax.Precision.HIGHEST
    proj_f = jnp.dot(h, Wef_ref[...], precision=prec)
    f = jax.nn.sigmoid(jnp.sum(curr * proj_f, axis=1, keepdims=True))
    i_vec = jnp.dot(h, Wei_ref[...], precision=prec)
    new_vec = curr * (1.0 - f) + f * i_vec
    norm = jnp.sqrt(jnp.sum(new_vec * new_vec, axis=1, keepdims=True))
    new_vec_n = new_vec / (norm + EPS)
    upd = curr + em * (new_vec_n - curr)

    sel = onehot * em
    E2 = E + sel[:, :, None] * (upd - curr)[:, None, :]
    E2_ref[...] = E2.reshape(BB * NSLOT, ED)

    d = (edist_ref[...] + fin) * (1.0 - sel)
    d = d * (iota65 != 0).astype(jnp.float32)
    od_ref[...] = d

    curr_e = upd
    nc = null_ref[...] + em * (curr_e - null_ref[...])
    nc = nc + fin * (h - nc)
    onc_ref[...] = nc

    oet_ref[...] = jnp.dot(h, Wr_ref[...], precision=prec)
    proj_e = jnp.dot(h, Wec_ref[...], precision=prec)
    dot_ej = jnp.sum(E2 * proj_e[:, None, :], axis=2)

    iota66 = jax.lax.broadcasted_iota(jnp.int32, (BB, NLOOK), 1)
    onehot66 = (iota66 == e_idx1).astype(jnp.float32)
    look = look_ref[...]
    look_at = jnp.sum(look * onehot66, axis=1, keepdims=True)
    look_out = jnp.maximum(look, onehot66)
    olook_ref[...] = look_out
    new_e = (1.0 - look_at) * em
    onent_ref[...] = nent_ref[...] + new_e.astype(jnp.int32)

    oeidx = dot_ej - jnp.exp(d * lam)
    oeidx_ref[...] = jnp.where(look_out[:, :NSLOT] > 0.0, oeidx, -jnp.inf)

    sel2 = nc + em * (curr_e - nc)
    oelen_ref[...] = (jnp.dot(h, Wlh_ref[...], precision=prec)
                      + jnp.dot(sel2, Wle_ref[...], precision=prec)
                      + bL_ref[...])
    ox_ref[...] = (em * jnp.dot(curr_e, Wx_ref[...], precision=prec)
                   + (1.0 - em) * jnp.dot(nc, Wxn_ref[...], precision=prec))


@functools.partial(jax.jit, static_argnums=())
def kernel(h, E, n_entities, e_dists, null_context, e_idx_lookup, e_t, e_idx,
           e_len, final_tok, W_R, W_Ec, lambda_dist, W_L, b_L, W_Ef, W_Ei,
           W_X, W_Xn):
    del e_len
    G = B // BB

    E_flat = E.reshape(B * NSLOT, ED)
    look_f = e_idx_lookup.astype(jnp.float32)
    et2 = e_t.reshape(B, 1)
    eidx2 = e_idx.reshape(B, 1)
    fin2 = final_tok.reshape(B, 1).astype(jnp.int32)
    nent2 = n_entities.reshape(B, 1)
    lam2 = lambda_dist.reshape(1, 1)
    bL2 = b_L.reshape(1, MAX_LEN)

    row = lambda i: (i, 0)
    rep = lambda i: (0, 0)
    in_specs = [
        pl.BlockSpec((BB, HD), row),
        pl.BlockSpec((BB * NSLOT, ED), row),
        pl.BlockSpec((BB, 1), row),
        pl.BlockSpec((BB, NSLOT), row),
        pl.BlockSpec((BB, ED), row),
        pl.BlockSpec((BB, NLOOK), row),
        pl.BlockSpec((BB, 1), row),
        pl.BlockSpec((BB, 1), row),
        pl.BlockSpec((BB, 1), row),
        pl.BlockSpec((1, 1), rep),
        pl.BlockSpec((HD, 2), rep),
        pl.BlockSpec((HD, ED), rep),
        pl.BlockSpec((HD, MAX_LEN), rep),
        pl.BlockSpec((ED, MAX_LEN), rep),
        pl.BlockSpec((1, MAX_LEN), rep),
        pl.BlockSpec((HD, ED), rep),
        pl.BlockSpec((HD, ED), rep),
        pl.BlockSpec((ED, HD), rep),
        pl.BlockSpec((ED, HD), rep),
    ]
    out_specs = [
        pl.BlockSpec((BB, 2), row),
        pl.BlockSpec((BB, NSLOT), row),
        pl.BlockSpec((BB, MAX_LEN), row),
        pl.BlockSpec((BB, HD), row),
        pl.BlockSpec((BB * NSLOT, ED), row),
        pl.BlockSpec((BB, 1), row),
        pl.BlockSpec((BB, NSLOT), row),
        pl.BlockSpec((BB, ED), row),
        pl.BlockSpec((BB, NLOOK), row),
    ]
    out_shapes = [
        jax.ShapeDtypeStruct((B, 2), jnp.float32),
        jax.ShapeDtypeStruct((B, NSLOT), jnp.float32),
        jax.ShapeDtypeStruct((B, MAX_LEN), jnp.float32),
        jax.ShapeDtypeStruct((B, HD), jnp.float32),
        jax.ShapeDtypeStruct((B * NSLOT, ED), jnp.float32),
        jax.ShapeDtypeStruct((B, 1), jnp.int32),
        jax.ShapeDtypeStruct((B, NSLOT), jnp.float32),
        jax.ShapeDtypeStruct((B, ED), jnp.float32),
        jax.ShapeDtypeStruct((B, NLOOK), jnp.float32),
    ]

    outs = pl.pallas_call(
        _body,
        grid=(G,),
        in_specs=in_specs,
        out_specs=out_specs,
        out_shape=out_shapes,
    )(h, E_flat, nent2, e_dists, null_context, look_f, et2, eidx2, fin2,
      lam2, W_R.T, W_Ec.T, W_L.T[:HD], W_L.T[HD:], bL2, W_Ef.T, W_Ei.T,
      W_X.T, W_Xn.T)

    (out_e_t, out_e_idx, out_e_len, out_x, E2_flat, n_ent2, d_out, nc_out,
     look_out_f) = outs
    return (out_e_t, out_e_idx, out_e_len, out_x,
            E2_flat.reshape(B, NSLOT, ED), n_ent2.reshape(B),
            d_out, nc_out, look_out_f.astype(bool))


# speedup vs baseline: 1.7948x; 1.7948x over previous
import functools

import jax
import jax.numpy as jnp
from jax import lax
from jax.experimental import pallas as pl
from jax.experimental.pallas import tpu as pltpu
from jax.experimental.pallas import tpu_sc as plsc

B = 4096
HD = 256
ED = 256
NSLOT = 65
NLOOK = 66
MAX_LEN = 25
EPS = 1e-20

SC_NC = 2
SC_NS = 16
SC_NW = SC_NC * SC_NS
BPW = B // SC_NW

BBA = 512
BBB = 128


def _sc_gather_body(E_hbm, eidx_hbm, out_hbm, eidx_v, rows_v, buf_v, sem):
    wid = lax.axis_index("s") * SC_NC + lax.axis_index("c")
    base = wid * BPW
    pltpu.sync_copy(eidx_hbm.at[pl.ds(base, BPW)], eidx_v)
    for k in range(BPW // 16):
        v = eidx_v[pl.ds(k * 16, 16)]
        rows_v[pl.ds(k * 16, 16)] = (
            (base + k * 16 + lax.iota(jnp.int32, 16)) * NSLOT + v + 1)
    pltpu.async_copy(E_hbm.at[rows_v], buf_v, sem).wait()
    pltpu.sync_copy(buf_v, out_hbm.at[pl.ds(base, BPW)])


def _gather_curr_sc(E_flat, e_idx):
    mesh = plsc.VectorSubcoreMesh(core_axis_name="c", subcore_axis_name="s")
    return pl.kernel(
        _sc_gather_body,
        mesh=mesh,
        out_type=jax.ShapeDtypeStruct((B, ED), jnp.float32),
        scratch_types=[
            pltpu.VMEM((BPW,), jnp.int32),
            pltpu.VMEM((BPW,), jnp.int32),
            pltpu.VMEM((BPW, ED), jnp.float32),
            pltpu.SemaphoreType.DMA,
        ],
    )(E_flat, e_idx)


def _body_a(h_ref, curr_ref, nent_ref, edist_ref, null_ref, look_ref, et_ref,
            eidx_ref, fin_ref, lam_ref, Wr_ref, Wec_ref, Wlh_ref, Wle_ref,
            bL_ref, Wef_ref, Wei_ref, Wx_ref, Wxn_ref,
            oet_ref, oelen_ref, ox_ref, onc_ref, od_ref, onent_ref,
            olook_ref, oupd_ref, oproj_ref, oadj_ref):
    h = h_ref[...]
    curr = curr_ref[...]
    e_idx1 = eidx_ref[...] + 1
    em = (et_ref[...] == 1).astype(jnp.float32)
    fin = fin_ref[...].astype(jnp.float32)
    lam = lam_ref[0, 0]

    prec = lax.Precision.HIGHEST
    proj_f = jnp.dot(h, Wef_ref[...], precision=prec)
    f = jax.nn.sigmoid(jnp.sum(curr * proj_f, axis=1, keepdims=True))
    i_vec = jnp.dot(h, Wei_ref[...], precision=prec)
    new_vec = curr * (1.0 - f) + f * i_vec
    norm = jnp.sqrt(jnp.sum(new_vec * new_vec, axis=1, keepdims=True))
    new_vec_n = new_vec / (norm + EPS)
    upd = curr + em * (new_vec_n - curr)
    oupd_ref[...] = upd

    iota65 = lax.broadcasted_iota(jnp.int32, (BBA, NSLOT), 1)
    onehot = (iota65 == e_idx1).astype(jnp.float32)

    d = (edist_ref[...] + fin) * (1.0 - onehot * em)
    d = d * (iota65 != 0).astype(jnp.float32)
    od_ref[...] = d

    nc = null_ref[...] + em * (upd - null_ref[...])
    nc = nc + fin * (h - nc)
    onc_ref[...] = nc

    iota66 = lax.broadcasted_iota(jnp.int32, (BBA, NLOOK), 1)
    onehot66 = (iota66 == e_idx1).astype(jnp.float32)
    look = look_ref[...]
    look_at = jnp.sum(look * onehot66, axis=1, keepdims=True)
    look_out = jnp.maximum(look, onehot66)
    olook_ref[...] = look_out
    onent_ref[...] = nent_ref[...] + ((1.0 - look_at) * em).astype(jnp.int32)

    proj_e = jnp.dot(h, Wec_ref[...], precision=prec)
    oproj_ref[...] = proj_e
    corr = jnp.sum((upd - curr) * proj_e, axis=1, keepdims=True)
    adj = -jnp.exp(d * lam) + onehot * corr
    oadj_ref[...] = jnp.where(look_out[:, :NSLOT] > 0.0, adj, -jnp.inf)

    oet_ref[...] = jnp.dot(h, Wr_ref[...], precision=prec)
    sel2 = nc + em * (upd - nc)
    oelen_ref[...] = (jnp.dot(h, Wlh_ref[...], precision=prec)
                      + jnp.dot(sel2, Wle_ref[...], precision=prec)
                      + bL_ref[...])
    ox_ref[...] = (em * jnp.dot(upd, Wx_ref[...], precision=prec)
                   + (1.0 - em) * jnp.dot(nc, Wxn_ref[...], precision=prec))


def _body_b(eidx_smem, E_ref, upd_ref, proj_ref, adj_ref, E2_ref, oeidx_ref):
    i = pl.program_id(0)
    E3 = E_ref[...]
    E2_ref[...] = E3
    proj = proj_ref[...]
    D2 = jnp.sum(E3 * proj[:, None, :], axis=2)
    oeidx_ref[...] = D2 + adj_ref[...]

    def scatter_one(s, carry):
        row = eidx_smem[i * BBB + s] + 1
        E2_ref[pl.ds(s, 1), pl.ds(row, 1), :] = (
            upd_ref[pl.ds(s, 1), :].reshape(1, 1, ED))
        return carry

    lax.fori_loop(0, BBB, scatter_one, 0)


@functools.partial(jax.jit, static_argnums=())
def kernel(h, E, n_entities, e_dists, null_context, e_idx_lookup, e_t, e_idx,
           e_len, final_tok, W_R, W_Ec, lambda_dist, W_L, b_L, W_Ef, W_Ei,
           W_X, W_Xn):
    del e_len

    E_flat = E.reshape(B * NSLOT, ED)
    curr = _gather_curr_sc(E_flat, e_idx)

    look_f = e_idx_lookup.astype(jnp.float32)
    et2 = e_t.reshape(B, 1)
    eidx2 = e_idx.reshape(B, 1)
    fin2 = final_tok.reshape(B, 1).astype(jnp.int32)
    nent2 = n_entities.reshape(B, 1)
    lam2 = lambda_dist.reshape(1, 1)
    bL2 = b_L.reshape(1, MAX_LEN)

    row = lambda i: (i, 0)
    rep = lambda i: (0, 0)
    GA = B // BBA
    outs_a = pl.pallas_call(
        _body_a,
        grid=(GA,),
        in_specs=[
            pl.BlockSpec((BBA, HD), row),
            pl.BlockSpec((BBA, ED), row),
            pl.BlockSpec((BBA, 1), row),
            pl.BlockSpec((BBA, NSLOT), row),
            pl.BlockSpec((BBA, ED), row),
            pl.BlockSpec((BBA, NLOOK), row),
            pl.BlockSpec((BBA, 1), row),
            pl.BlockSpec((BBA, 1), row),
            pl.BlockSpec((BBA, 1), row),
            pl.BlockSpec((1, 1), rep),
            pl.BlockSpec((HD, 2), rep),
            pl.BlockSpec((HD, ED), rep),
            pl.BlockSpec((HD, MAX_LEN), rep),
            pl.BlockSpec((ED, MAX_LEN), rep),
            pl.BlockSpec((1, MAX_LEN), rep),
            pl.BlockSpec((HD, ED), rep),
            pl.BlockSpec((HD, ED), rep),
            pl.BlockSpec((ED, HD), rep),
            pl.BlockSpec((ED, HD), rep),
        ],
        out_specs=[
            pl.BlockSpec((BBA, 2), row),
            pl.BlockSpec((BBA, MAX_LEN), row),
            pl.BlockSpec((BBA, HD), row),
            pl.BlockSpec((BBA, ED), row),
            pl.BlockSpec((BBA, NSLOT), row),
            pl.BlockSpec((BBA, 1), row),
            pl.BlockSpec((BBA, NLOOK), row),
            pl.BlockSpec((BBA, ED), row),
            pl.BlockSpec((BBA, ED), row),
            pl.BlockSpec((BBA, NSLOT), row),
        ],
        out_shape=[
            jax.ShapeDtypeStruct((B, 2), jnp.float32),
            jax.ShapeDtypeStruct((B, MAX_LEN), jnp.float32),
            jax.ShapeDtypeStruct((B, HD), jnp.float32),
            jax.ShapeDtypeStruct((B, ED), jnp.float32),
            jax.ShapeDtypeStruct((B, NSLOT), jnp.float32),
            jax.ShapeDtypeStruct((B, 1), jnp.int32),
            jax.ShapeDtypeStruct((B, NLOOK), jnp.float32),
            jax.ShapeDtypeStruct((B, ED), jnp.float32),
            jax.ShapeDtypeStruct((B, ED), jnp.float32),
            jax.ShapeDtypeStruct((B, NSLOT), jnp.float32),
        ],
    )(h, curr, nent2, e_dists, null_context, look_f, et2, eidx2, fin2,
      lam2, W_R.T, W_Ec.T, W_L.T[:HD], W_L.T[HD:], bL2, W_Ef.T, W_Ei.T,
      W_X.T, W_Xn.T)

    (out_e_t, out_e_len, out_x, nc_out, d_out, n_ent2, look_out_f, upd,
     proj_e, adj) = outs_a

    GB = B // BBB
    row3 = lambda i, *_: (i, 0, 0)
    row2 = lambda i, *_: (i, 0)
    E2, out_e_idx = pl.pallas_call(
        _body_b,
        grid_spec=pltpu.PrefetchScalarGridSpec(
            num_scalar_prefetch=1,
            grid=(GB,),
            in_specs=[
                pl.BlockSpec((BBB, NSLOT, ED), row3),
                pl.BlockSpec((BBB, ED), row2),
                pl.BlockSpec((BBB, ED), row2),
                pl.BlockSpec((BBB, NSLOT), row2),
            ],
            out_specs=[
                pl.BlockSpec((BBB, NSLOT, ED), row3),
                pl.BlockSpec((BBB, NSLOT), row2),
            ],
        ),
        out_shape=[
            jax.ShapeDtypeStruct((B, NSLOT, ED), jnp.float32),
            jax.ShapeDtypeStruct((B, NSLOT), jnp.float32),
        ],
    )(e_idx, E, upd, proj_e, adj)

    return (out_e_t, out_e_idx, out_e_len, out_x, E2, n_ent2.reshape(B),
            d_out, nc_out, look_out_f.astype(bool))


# speedup vs baseline: 2.3600x; 1.3149x over previous
import functools

import jax
import jax.numpy as jnp
from jax import lax
from jax.experimental import pallas as pl
from jax.experimental.pallas import tpu as pltpu

B = 4096
HD = 256
ED = 256
NSLOT = 65
NLOOK = 66
MAX_LEN = 25
EPS = 1e-20

BBA = 512
BBB = 128
BBC = 128


def _body_b(eidx_smem, E_ref, h_ref, Wec_ref,
            E2_ref, d2_ref, curr_ref, proj_ref):
    i = pl.program_id(0)
    E3 = E_ref[...]
    E2_ref[...] = E3
    proj = jnp.dot(h_ref[...], Wec_ref[...], precision=lax.Precision.HIGHEST)
    proj_ref[...] = proj
    d2_ref[...] = jnp.sum(E3 * proj[:, None, :], axis=2)

    def gather_one(s, carry):
        row = eidx_smem[i * BBB + s] + 1
        curr_ref[pl.ds(s, 1), :] = (
            E_ref[pl.ds(s, 1), pl.ds(row, 1), :].reshape(1, ED))
        return carry

    lax.fori_loop(0, BBB, gather_one, 0)


def _body_a(h_ref, curr_ref, d2_ref, nent_ref, edist_ref, null_ref, look_ref,
            et_ref, eidx_ref, fin_ref, lam_ref, Wr_ref, Wec_unused_ref,
            Wlh_ref, Wle_ref, bL_ref, Wef_ref, Wei_ref, Wx_ref, Wxn_ref,
            proj_ref,
            oet_ref, oeidx_ref, oelen_ref, ox_ref, onc_ref, od_ref,
            onent_ref, olook_ref, oupd_ref):
    del Wec_unused_ref
    h = h_ref[...]
    curr = curr_ref[...]
    e_idx1 = eidx_ref[...] + 1
    em = (et_ref[...] == 1).astype(jnp.float32)
    fin = fin_ref[...].astype(jnp.float32)
    lam = lam_ref[0, 0]

    prec = lax.Precision.HIGHEST
    proj_f = jnp.dot(h, Wef_ref[...], precision=prec)
    f = jax.nn.sigmoid(jnp.sum(curr * proj_f, axis=1, keepdims=True))
    i_vec = jnp.dot(h, Wei_ref[...], precision=prec)
    new_vec = curr * (1.0 - f) + f * i_vec
    norm = jnp.sqrt(jnp.sum(new_vec * new_vec, axis=1, keepdims=True))
    new_vec_n = new_vec / (norm + EPS)
    upd = curr + em * (new_vec_n - curr)
    oupd_ref[...] = upd

    iota65 = lax.broadcasted_iota(jnp.int32, (BBA, NSLOT), 1)
    onehot = (iota65 == e_idx1).astype(jnp.float32)

    d = (edist_ref[...] + fin) * (1.0 - onehot * em)
    d = d * (iota65 != 0).astype(jnp.float32)
    od_ref[...] = d

    nc = null_ref[...] + em * (upd - null_ref[...])
    nc = nc + fin * (h - nc)
    onc_ref[...] = nc

    iota66 = lax.broadcasted_iota(jnp.int32, (BBA, NLOOK), 1)
    onehot66 = (iota66 == e_idx1).astype(jnp.float32)
    look = look_ref[...]
    look_at = jnp.sum(look * onehot66, axis=1, keepdims=True)
    look_out = jnp.maximum(look, onehot66)
    olook_ref[...] = look_out
    onent_ref[...] = nent_ref[...] + ((1.0 - look_at) * em).astype(jnp.int32)

    proj_e = proj_ref[...]
    corr = jnp.sum((upd - curr) * proj_e, axis=1, keepdims=True)
    scores = d2_ref[...] - jnp.exp(d * lam) + onehot * corr
    oeidx_ref[...] = jnp.where(look_out[:, :NSLOT] > 0.0, scores, -jnp.inf)

    oet_ref[...] = jnp.dot(h, Wr_ref[...], precision=prec)
    sel2 = nc + em * (upd - nc)
    oelen_ref[...] = (jnp.dot(h, Wlh_ref[...], precision=prec)
                      + jnp.dot(sel2, Wle_ref[...], precision=prec)
                      + bL_ref[...])
    ox_ref[...] = (em * jnp.dot(upd, Wx_ref[...], precision=prec)
                   + (1.0 - em) * jnp.dot(nc, Wxn_ref[...], precision=prec))


def _body_c(eidx_smem, upd_ref, e2in_ref, e2out_ref, sem):
    del e2in_ref
    i = pl.program_id(0)

    def fire(s, carry):
        g = i * BBC + s
        row = eidx_smem[g] + 1
        pltpu.make_async_copy(upd_ref.at[s], e2out_ref.at[g, row], sem).start()
        return carry

    lax.fori_loop(0, BBC, fire, 0)

    def drain(s, carry):
        g = i * BBC + s
        row = eidx_smem[g] + 1
        pltpu.make_async_copy(upd_ref.at[s], e2out_ref.at[g, row], sem).wait()
        return carry

    lax.fori_loop(0, BBC, drain, 0)


@functools.partial(jax.jit, static_argnums=())
def kernel(h, E, n_entities, e_dists, null_context, e_idx_lookup, e_t, e_idx,
           e_len, final_tok, W_R, W_Ec, lambda_dist, W_L, b_L, W_Ef, W_Ei,
           W_X, W_Xn):
    del e_len

    look_f = e_idx_lookup.astype(jnp.float32)
    et2 = e_t.reshape(B, 1)
    eidx2 = e_idx.reshape(B, 1)
    fin2 = final_tok.reshape(B, 1).astype(jnp.int32)
    nent2 = n_entities.reshape(B, 1)
    lam2 = lambda_dist.reshape(1, 1)
    bL2 = b_L.reshape(1, MAX_LEN)

    row2 = lambda i, *_: (i, 0)
    row3 = lambda i, *_: (i, 0, 0)
    rep = lambda i, *_: (0, 0)

    GB = B // BBB
    E2_raw, d2_old, curr, proj_e = pl.pallas_call(
        _body_b,
        grid_spec=pltpu.PrefetchScalarGridSpec(
            num_scalar_prefetch=1,
            grid=(GB,),
            in_specs=[
                pl.BlockSpec((BBB, NSLOT, ED), row3),
                pl.BlockSpec((BBB, HD), row2),
                pl.BlockSpec((HD, ED), rep),
            ],
            out_specs=[
                pl.BlockSpec((BBB, NSLOT, ED), row3),
                pl.BlockSpec((BBB, NSLOT), row2),
                pl.BlockSpec((BBB, ED), row2),
                pl.BlockSpec((BBB, ED), row2),
            ],
        ),
        out_shape=[
            jax.ShapeDtypeStruct((B, NSLOT, ED), jnp.float32),
            jax.ShapeDtypeStruct((B, NSLOT), jnp.float32),
            jax.ShapeDtypeStruct((B, ED), jnp.float32),
            jax.ShapeDtypeStruct((B, ED), jnp.float32),
        ],
    )(e_idx, E, h, W_Ec.T)

    GA = B // BBA
    outs_a = pl.pallas_call(
        _body_a,
        grid=(GA,),
        in_specs=[
            pl.BlockSpec((BBA, HD), row2),
            pl.BlockSpec((BBA, ED), row2),
            pl.BlockSpec((BBA, NSLOT), row2),
            pl.BlockSpec((BBA, 1), row2),
            pl.BlockSpec((BBA, NSLOT), row2),
            pl.BlockSpec((BBA, ED), row2),
            pl.BlockSpec((BBA, NLOOK), row2),
            pl.BlockSpec((BBA, 1), row2),
            pl.BlockSpec((BBA, 1), row2),
            pl.BlockSpec((BBA, 1), row2),
            pl.BlockSpec((1, 1), rep),
            pl.BlockSpec((HD, 2), rep),
            pl.BlockSpec((HD, ED), rep),
            pl.BlockSpec((HD, MAX_LEN), rep),
            pl.BlockSpec((ED, MAX_LEN), rep),
            pl.BlockSpec((1, MAX_LEN), rep),
            pl.BlockSpec((HD, ED), rep),
            pl.BlockSpec((HD, ED), rep),
            pl.BlockSpec((ED, HD), rep),
            pl.BlockSpec((ED, HD), rep),
            pl.BlockSpec((BBA, ED), row2),
        ],
        out_specs=[
            pl.BlockSpec((BBA, 2), row2),
            pl.BlockSpec((BBA, NSLOT), row2),
            pl.BlockSpec((BBA, MAX_LEN), row2),
            pl.BlockSpec((BBA, HD), row2),
            pl.BlockSpec((BBA, ED), row2),
            pl.BlockSpec((BBA, NSLOT), row2),
            pl.BlockSpec((BBA, 1), row2),
            pl.BlockSpec((BBA, NLOOK), row2),
            pl.BlockSpec((BBA, ED), row2),
        ],
        out_shape=[
            jax.ShapeDtypeStruct((B, 2), jnp.float32),
            jax.ShapeDtypeStruct((B, NSLOT), jnp.float32),
            jax.ShapeDtypeStruct((B, MAX_LEN), jnp.float32),
            jax.ShapeDtypeStruct((B, HD), jnp.float32),
            jax.ShapeDtypeStruct((B, ED), jnp.float32),
            jax.ShapeDtypeStruct((B, NSLOT), jnp.float32),
            jax.ShapeDtypeStruct((B, 1), jnp.int32),
            jax.ShapeDtypeStruct((B, NLOOK), jnp.float32),
            jax.ShapeDtypeStruct((B, ED), jnp.float32),
        ],
    )(h, curr, d2_old, nent2, e_dists, null_context, look_f, et2, eidx2,
      fin2, lam2, W_R.T, W_Ec.T, W_L.T[:HD], W_L.T[HD:], bL2, W_Ef.T,
      W_Ei.T, W_X.T, W_Xn.T, proj_e)

    (out_e_t, out_e_idx, out_e_len, out_x, nc_out, d_out, n_ent2,
     look_out_f, upd) = outs_a

    GC = B // BBC
    E2 = pl.pallas_call(
        _body_c,
        grid_spec=pltpu.PrefetchScalarGridSpec(
            num_scalar_prefetch=1,
            grid=(GC,),
            in_specs=[
                pl.BlockSpec((BBC, ED), row2),
                pl.BlockSpec(memory_space=pl.ANY),
            ],
            out_specs=[
                pl.BlockSpec(memory_space=pl.ANY),
            ],
            scratch_shapes=[pltpu.SemaphoreType.DMA],
        ),
        out_shape=[jax.ShapeDtypeStruct((B, NSLOT, ED), jnp.float32)],
        input_output_aliases={2: 0},
    )(e_idx, upd, E2_raw)
    E2 = E2[0]

    return (out_e_t, out_e_idx, out_e_len, out_x, E2, n_ent2.reshape(B),
            d_out, nc_out, look_out_f.astype(bool))
